# baseline (speedup 1.0000x reference)
import functools

import jax
import jax.numpy as jnp
from jax import lax
from jax.experimental import pallas as pl
from jax.experimental.pallas import tpu as pltpu
from jax.experimental.pallas import tpu_sc as plsc

_WINDOW = 1000
_STRIDE = 200
_HGRP = 2


def _body(num_new, x_hbm, labels_hbm, out6, lbl_hbm,
          r0, r1, f0, f1, lblv, lblb, gs0, gs1, os0, os1):
    w = lax.axis_index("c") * 16 + lax.axis_index("s")
    num_samples = x_hbm.shape[0]
    h = x_hbm.shape[2]

    pltpu.sync_copy(labels_hbm, lblv)
    chunk_id = jnp.zeros((16,), jnp.int32) + (w // 16)
    chunk = jnp.zeros((16,), jnp.float32)
    for i in range(num_samples // 16):
        chunk = jnp.where(chunk_id == i, lblv[pl.ds(16 * i, 16)], chunk)
    lanes = lax.iota(jnp.int32, 16)
    val = jnp.sum(jnp.where(lanes == (w % 16), chunk, 0.0))
    lblb[...] = jnp.zeros((16,), jnp.float32) + val
    pltpu.sync_copy(lblb, lbl_hbm.at[pl.ds(w * num_new, num_new)])

    c = w // 16
    h0 = (w % 16) * _HGRP

    rows = (r0, r1)
    fbufs = (f0, f1)
    gsems = (gs0, gs1)
    osems = (os0, os1)
    jlane = lax.iota(jnp.int32, 16)

    def fire_stage(n, b):
        return pltpu.async_copy(
            x_hbm.at[n, c, pl.ds(h0, _HGRP), :, :],
            rows[b].at[:, :, pl.ds(0, _STRIDE)], gsems[b])

    def dst_slice(n):
        return out6.at[c, pl.ds(h0, _HGRP), :, n // 8, :,
                       pl.ds((n % 8) * 16, 16)]

    def transpose_task(b):
        rb, fb = rows[b], fbufs[b]
        nchunk = _WINDOW // _STRIDE
        for hh in range(_HGRP):
            ihh = jnp.zeros((16,), jnp.int32) + hh
            for a in range(nchunk):
                ichunk = jlane + a
                tq0 = a * (_STRIDE // 8)

                def rq_body(rq):
                    r0 = rq * 8
                    vecs = [plsc.load_gather(
                        rb, [ihh, ichunk, jnp.zeros((16,), jnp.int32)
                             + (r0 + k)]) for k in range(8)]
                    for k in range(8):
                        fb[hh, tq0 + rq, k, :] = vecs[k]

                plsc.parallel_loop(0, _STRIDE // 8, unroll=4)(rq_body)

    fire_stage(0, 0)

    def task_body(i):
        for b in range(2):
            n = 2 * i + b
            pltpu.make_async_copy(
                x_hbm.at[n, c, pl.ds(h0, _HGRP), :, :],
                rows[b].at[:, :, pl.ds(0, _STRIDE)], gsems[b]).wait()

            @pl.when(n < num_samples - 1)
            def _():
                fire_stage(n + 1, 1 - b)
                return None

            @pl.when(n >= 2)
            def _():
                pltpu.make_async_copy(fbufs[b], dst_slice(n - 2),
                                      osems[b]).wait()

            transpose_task(b)
            pltpu.async_copy(fbufs[b], dst_slice(n), osems[b])

    pl.loop(0, num_samples // 2)(task_body)

    for b in range(2):
        pltpu.make_async_copy(fbufs[b], dst_slice(num_samples - 2 + b),
                              osems[b]).wait()


def kernel(x, labels):
    num_samples, channels, h, w = x.shape
    num_new = (w - _WINDOW) // _STRIDE + 1
    x5 = x.reshape(num_samples, channels, h, w // _STRIDE, _STRIDE)
    mesh = plsc.VectorSubcoreMesh(core_axis_name="c", subcore_axis_name="s")
    out_type = (
        jax.ShapeDtypeStruct(
            (channels, h, _WINDOW // 8, (num_samples * num_new) // 128, 8, 128),
            x.dtype),
        jax.ShapeDtypeStruct((num_samples * num_new,), labels.dtype),
    )
    f = pl.kernel(
        functools.partial(_body, num_new),
        out_type=out_type,
        mesh=mesh,
        compiler_params=pltpu.CompilerParams(use_tc_tiling_on_sc=False,
                                             needs_layout_passes=False),
        scratch_types=[
            pltpu.VMEM((_HGRP, w // _STRIDE, _STRIDE + 1), jnp.float32),
            pltpu.VMEM((_HGRP, w // _STRIDE, _STRIDE + 1), jnp.float32),
            pltpu.VMEM((_HGRP, _WINDOW // 8, 8, 16), jnp.float32),
            pltpu.VMEM((_HGRP, _WINDOW // 8, 8, 16), jnp.float32),
            pltpu.VMEM((num_samples,), jnp.float32),
            pltpu.VMEM((16,), jnp.float32),
            pltpu.SemaphoreType.DMA,
            pltpu.SemaphoreType.DMA,
            pltpu.SemaphoreType.DMA,
            pltpu.SemaphoreType.DMA,
        ],
    )
    o6, lbl = f(x5, labels)
    out = o6.transpose(3, 5, 0, 1, 2, 4).reshape(
        num_samples * num_new, channels, h, _WINDOW)
    return out, lbl


# speedup vs baseline: 1.9633x; 1.9633x over previous
import functools

import jax
import jax.numpy as jnp
from jax import lax
from jax.experimental import pallas as pl
from jax.experimental.pallas import tpu as pltpu
from jax.experimental.pallas import tpu_sc as plsc

_WINDOW = 1000
_STRIDE = 200
_PAIR = 2


def _body(num_new, x_hbm, labels_hbm, out6, lbl_hbm,
          r0, r1, f0, f1, lblv, lblb, gs0, gs1, os0, os1):
    w = lax.axis_index("c") * 16 + lax.axis_index("s")
    num_samples = x_hbm.shape[0]
    h = x_hbm.shape[2]

    pltpu.sync_copy(labels_hbm, lblv)
    chunk_id = jnp.zeros((16,), jnp.int32) + (w // 16)
    chunk = jnp.zeros((16,), jnp.float32)
    for i in range(num_samples // 16):
        chunk = jnp.where(chunk_id == i, lblv[pl.ds(16 * i, 16)], chunk)
    lanes = lax.iota(jnp.int32, 16)
    val = jnp.sum(jnp.where(lanes == (w % 16), chunk, 0.0))
    lblb[...] = jnp.zeros((16,), jnp.float32) + val
    pltpu.sync_copy(lblb, lbl_hbm.at[pl.ds(w * num_new, num_new)])

    pairs_per_plane = num_samples // _PAIR

    def task_coords(k):
        p = jnp.where(k < pairs_per_plane, w, w + 32)
        c = p // h
        hc = p % h
        n = _PAIR * (k % pairs_per_plane)
        return c, hc, n

    rows = (r0, r1)
    fbufs = (f0, f1)
    gsems = (gs0, gs1)
    osems = (os0, os1)
    jlane = lax.iota(jnp.int32, 16)
    base = _STRIDE * jlane

    def stage_refs(k):
        c, hc, n = task_coords(k)
        return x_hbm.at[pl.ds(n, _PAIR), c, hc, :]

    def dst_slice(k):
        c, hc, n = task_coords(k)
        return out6.at[c, hc, :, n // 8, :, pl.ds((n % 8) * 16, 16 * _PAIR)]

    def transpose_task(b):
        rb, fb = rows[b], fbufs[b]
        for nn in range(_PAIR):
            inn = jnp.zeros((16,), jnp.int32) + nn

            def t_body(tq):
                t0 = tq * 8
                vecs = [plsc.load_gather(rb, [inn, base + (t0 + k)])
                        for k in range(8)]
                for k in range(8):
                    fb[tq, k, pl.ds(16 * nn, 16)] = vecs[k]

            plsc.parallel_loop(0, _WINDOW // 8, unroll=4)(t_body)

    num_tasks = 2 * pairs_per_plane
    pltpu.async_copy(stage_refs(jnp.int32(0)), rows[0], gsems[0])

    def task_body(i):
        for b in range(2):
            k = 2 * i + b
            pltpu.make_async_copy(stage_refs(k), rows[b], gsems[b]).wait()

            @pl.when(k < num_tasks - 1)
            def _():
                pltpu.async_copy(stage_refs(k + 1), rows[1 - b], gsems[1 - b])
                return None

            @pl.when(k >= 2)
            def _():
                pltpu.make_async_copy(fbufs[b], dst_slice(k - 2),
                                      osems[b]).wait()

            transpose_task(b)
            pltpu.async_copy(fbufs[b], dst_slice(k), osems[b])

    pl.loop(0, num_tasks // 2)(task_body)

    for b in range(2):
        pltpu.make_async_copy(fbufs[b], dst_slice(num_tasks - 2 + b),
                              osems[b]).wait()


def kernel(x, labels):
    num_samples, channels, h, w = x.shape
    num_new = (w - _WINDOW) // _STRIDE + 1
    mesh = plsc.VectorSubcoreMesh(core_axis_name="c", subcore_axis_name="s")
    out_type = (
        jax.ShapeDtypeStruct(
            (channels, h, _WINDOW // 8, (num_samples * num_new) // 128, 8, 128),
            x.dtype),
        jax.ShapeDtypeStruct((num_samples * num_new,), labels.dtype),
    )
    f = pl.kernel(
        functools.partial(_body, num_new),
        out_type=out_type,
        mesh=mesh,
        compiler_params=pltpu.CompilerParams(use_tc_tiling_on_sc=False,
                                             needs_layout_passes=False),
        scratch_types=[
            pltpu.VMEM((_PAIR, w), jnp.float32),
            pltpu.VMEM((_PAIR, w), jnp.float32),
            pltpu.VMEM((_WINDOW // 8, 8, 16 * _PAIR), jnp.float32),
            pltpu.VMEM((_WINDOW // 8, 8, 16 * _PAIR), jnp.float32),
            pltpu.VMEM((num_samples,), jnp.float32),
            pltpu.VMEM((16,), jnp.float32),
            pltpu.SemaphoreType.DMA,
            pltpu.SemaphoreType.DMA,
            pltpu.SemaphoreType.DMA,
            pltpu.SemaphoreType.DMA,
        ],
    )
    o6, lbl = f(x, labels)
    out = o6.transpose(3, 5, 0, 1, 2, 4).reshape(
        num_samples * num_new, channels, h, _WINDOW)
    return out, lbl
